# baseline (speedup 1.0000x reference)
import functools

import jax
import jax.numpy as jnp
from jax import lax
from jax.experimental import pallas as pl
from jax.experimental.pallas import tpu as pltpu
from jax.experimental.pallas import tpu_sc as plsc

_N = 10000
_D = 128
_E = 320000

_NC = 2
_NS = 16
_NW = _NC * _NS

_N_TAB = 10016
_N_PAD = 10112
_EPW = 10240
_E_PAD = _EPW * _NW
_CHUNK = 64
_NCHUNK = _EPW // _CHUNK
_IBLK = 16
_NBLK = _NCHUNK // _IBLK
_ROWS_OUT = 624
_ZROWS = _N_PAD // _NS


def _mlp_body(x_ref, w_ref, b_ref, a_ref, c_ref):
    xb = x_ref[...]
    w = w_ref[...]
    a_ref[...] = jnp.dot(xb, w[:_D, :], preferred_element_type=jnp.float32) + b_ref[...]
    c_ref[...] = jnp.dot(xb, w[_D:, :], preferred_element_type=jnp.float32)


def _mlp(x, W, b2d):
    blk = 1000
    return pl.pallas_call(
        _mlp_body,
        grid=(_N // blk,),
        in_specs=[pl.BlockSpec((blk, _D), lambda i: (i, 0)),
                  pl.BlockSpec((2 * _D, _D), lambda i: (0, 0)),
                  pl.BlockSpec((1, _D), lambda i: (0, 0))],
        out_specs=[pl.BlockSpec((blk, _D), lambda i: (i, 0)),
                   pl.BlockSpec((blk, _D), lambda i: (i, 0))],
        out_shape=[jax.ShapeDtypeStruct((_N, _D), jnp.float32),
                   jax.ShapeDtypeStruct((_N, _D), jnp.float32)],
    )(x, W, b2d)


def _writeback(src_sh, dst_hbm, c, s):
    obase = s * _ROWS_OUT

    @pl.when(s < _NS - 1)
    def _():
        pltpu.sync_copy(src_sh.at[pl.ds(obase, _ROWS_OUT)],
                        dst_hbm.at[c, pl.ds(obase, _ROWS_OUT)])

    @pl.when(s == _NS - 1)
    def _():
        last = _N - (_NS - 1) * _ROWS_OUT
        lbase = (_NS - 1) * _ROWS_OUT
        pltpu.sync_copy(src_sh.at[pl.ds(lbase, last)],
                        dst_hbm.at[c, pl.ds(lbase, last)])


def _edge_body(a_hbm, b_hbm, src_hbm, dst_hbm, vals_out,
               sidx, didx, rows, rowsb, semi, sema, semb, acc):
    c = lax.axis_index("c")
    s = lax.axis_index("s")
    wid = s * _NC + c

    zero16 = jnp.zeros((16,), jnp.float32)

    def zrow(r, carry):
        for j in range(_D // 16):
            rows[0, r, pl.ds(j * 16, 16)] = zero16
        return carry
    lax.fori_loop(0, _CHUNK, zrow, 0)

    zbase = s * _ZROWS
    for z in range(_ZROWS // _CHUNK):
        pltpu.sync_copy(rows.at[0, pl.ds(0, _CHUNK)],
                        acc.at[pl.ds(zbase + z * _CHUNK, _CHUNK)])
    ztail = _ZROWS % _CHUNK
    if ztail:
        zoff = zbase + (_ZROWS // _CHUNK) * _CHUNK
        pltpu.sync_copy(rows.at[0, pl.ds(0, ztail)], acc.at[pl.ds(zoff, ztail)])
    plsc.subcore_barrier()

    erow = wid * _EPW

    _IB = _IBLK * _CHUNK

    def start_idx(bb, ibs):
        e0 = erow + bb * _IB
        pltpu.async_copy(src_hbm.at[pl.ds(e0, _IB)], sidx.at[ibs], semi.at[ibs])
        for j in range(_IBLK):
            pltpu.async_copy(dst_hbm.at[pl.ds(e0 + j * _CHUNK, _CHUNK)],
                             didx.at[ibs * _IBLK + j], semi.at[ibs])

    def wait_idx(bb, ibs):
        e0 = erow + bb * _IB
        pltpu.make_async_copy(src_hbm.at[pl.ds(e0, _IB)], sidx.at[ibs],
                              semi.at[ibs]).wait()
        for j in range(_IBLK):
            pltpu.make_async_copy(dst_hbm.at[pl.ds(e0 + j * _CHUNK, _CHUNK)],
                                  didx.at[ibs * _IBLK + j], semi.at[ibs]).wait()

    def start_gather(ibs, j, slot):
        pltpu.async_copy(a_hbm.at[sidx.at[ibs, pl.ds(j * _CHUNK, _CHUNK)]],
                         rows.at[slot], sema.at[slot])
        pltpu.async_copy(b_hbm.at[didx.at[ibs * _IBLK + j]],
                         rowsb.at[slot], semb.at[slot])

    def wait_gather(ibs, j, slot):
        pltpu.make_async_copy(a_hbm.at[sidx.at[ibs, pl.ds(j * _CHUNK, _CHUNK)]],
                              rows.at[slot], sema.at[slot]).wait()
        pltpu.make_async_copy(b_hbm.at[didx.at[ibs * _IBLK + j]],
                              rowsb.at[slot], semb.at[slot]).wait()

    start_idx(0, 0)

    def block(bb, carry):
        ibs = lax.rem(bb, 2)
        wait_idx(bb, ibs)

        @pl.when(bb < _NBLK - 1)
        def _():
            start_idx(bb + 1, 1 - ibs)
        start_gather(ibs, 0, 0)

        def chunk(j, inner):
            slot = lax.rem(j, 2)

            @pl.when(j < _IBLK - 1)
            def _():
                start_gather(ibs, j + 1, 1 - slot)
            wait_gather(ibs, j, slot)

            def relu_row(r, rin):
                for jj in range(_D // 16):
                    sl = pl.ds(jj * 16, 16)
                    rows[slot, r, sl] = jnp.maximum(
                        rows[slot, r, sl] + rowsb[slot, r, sl], 0.0)
                return rin
            lax.fori_loop(0, _CHUNK, relu_row, 0)

            pltpu.sync_copy(rows.at[slot], acc.at[didx.at[ibs * _IBLK + j]],
                            add=True)
            return inner
        lax.fori_loop(0, _IBLK, chunk, 0)
        return carry
    lax.fori_loop(0, _NBLK, block, 0)

    plsc.subcore_barrier()
    _writeback(acc, vals_out, c, s)


_edge_call = functools.partial(
    pl.kernel,
    out_type=jax.ShapeDtypeStruct((_NC, _N, _D), jnp.float32),
    mesh=plsc.VectorSubcoreMesh(core_axis_name="c", subcore_axis_name="s"),
    scratch_types=[
        pltpu.VMEM((2, _IBLK * _CHUNK), jnp.int32),
        pltpu.VMEM((2 * _IBLK, _CHUNK), jnp.int32),
        pltpu.VMEM((2, _CHUNK, _D), jnp.float32),
        pltpu.VMEM((2, _CHUNK, _D), jnp.float32),
        pltpu.SemaphoreType.DMA((2,)),
        pltpu.SemaphoreType.DMA((2,)),
        pltpu.SemaphoreType.DMA((2,)),
        pltpu.VMEM_SHARED((_N_PAD, _D), jnp.float32),
    ],
)(_edge_body)


def _cnt_body(dst_hbm, cnt_out, didx, ones, semi, cacc):
    c = lax.axis_index("c")
    s = lax.axis_index("s")
    wid = s * _NC + c

    zero16 = jnp.zeros((16,), jnp.float32)
    ones16 = jnp.ones((16,), jnp.float32)

    def zone(r, carry):
        for j in range(_D // 16):
            ones[r, pl.ds(j * 16, 16)] = zero16
        return carry
    lax.fori_loop(0, _CHUNK, zone, 0)

    zbase = s * _ZROWS
    for z in range(_ZROWS // _CHUNK):
        pltpu.sync_copy(ones.at[pl.ds(0, _CHUNK)],
                        cacc.at[pl.ds(zbase + z * _CHUNK, _CHUNK)])
    ztail = _ZROWS % _CHUNK
    if ztail:
        zoff = zbase + (_ZROWS // _CHUNK) * _CHUNK
        pltpu.sync_copy(ones.at[pl.ds(0, ztail)], cacc.at[pl.ds(zoff, ztail)])

    def fone(r, carry):
        ones[r, pl.ds(0, 16)] = ones16
        return carry
    lax.fori_loop(0, _CHUNK, fone, 0)
    plsc.subcore_barrier()

    erow = wid * _EPW
    _IB = _IBLK * _CHUNK

    def start_idx(bb, ibs):
        e0 = erow + bb * _IB
        for j in range(_IBLK):
            pltpu.async_copy(dst_hbm.at[pl.ds(e0 + j * _CHUNK, _CHUNK)],
                             didx.at[ibs * _IBLK + j], semi.at[ibs])

    def wait_idx(bb, ibs):
        e0 = erow + bb * _IB
        for j in range(_IBLK):
            pltpu.make_async_copy(dst_hbm.at[pl.ds(e0 + j * _CHUNK, _CHUNK)],
                                  didx.at[ibs * _IBLK + j], semi.at[ibs]).wait()

    start_idx(0, 0)

    def block(bb, carry):
        ibs = lax.rem(bb, 2)
        wait_idx(bb, ibs)

        @pl.when(bb < _NBLK - 1)
        def _():
            start_idx(bb + 1, 1 - ibs)

        def chunk(j, inner):
            pltpu.sync_copy(ones, cacc.at[didx.at[ibs * _IBLK + j]], add=True)
            return inner
        lax.fori_loop(0, _IBLK, chunk, 0)
        return carry
    lax.fori_loop(0, _NBLK, block, 0)

    plsc.subcore_barrier()
    _writeback(cacc, cnt_out, c, s)


_cnt_call = functools.partial(
    pl.kernel,
    out_type=jax.ShapeDtypeStruct((_NC, _N, _D), jnp.float32),
    mesh=plsc.VectorSubcoreMesh(core_axis_name="c", subcore_axis_name="s"),
    scratch_types=[
        pltpu.VMEM((2 * _IBLK, _CHUNK), jnp.int32),
        pltpu.VMEM((_CHUNK, _D), jnp.float32),
        pltpu.SemaphoreType.DMA((2,)),
        pltpu.VMEM_SHARED((_N_PAD, _D), jnp.float32),
    ],
)(_cnt_body)


def _comb_body(pv_ref, pc_ref, out_ref):
    vals = pv_ref[0] + pv_ref[1]
    cnt = pc_ref[0, :, 0:1] + pc_ref[1, :, 0:1]
    out_ref[...] = vals / jnp.maximum(cnt, 1.0)


def _combine(pvals, pcnt):
    blk = 1000
    return pl.pallas_call(
        _comb_body,
        grid=(_N // blk,),
        in_specs=[pl.BlockSpec((_NC, blk, _D), lambda i: (0, i, 0)),
                  pl.BlockSpec((_NC, blk, _D), lambda i: (0, i, 0))],
        out_specs=pl.BlockSpec((blk, _D), lambda i: (i, 0)),
        out_shape=jax.ShapeDtypeStruct((_N, _D), jnp.float32),
    )(pvals, pcnt)


def kernel(x, edge_index, W, b):
    A, B = _mlp(x, W, b.reshape(1, _D))
    A = jnp.pad(A, ((0, _N_TAB - _N), (0, 0)))
    B = jnp.pad(B, ((0, _N_TAB - _N), (0, 0)))
    src = edge_index[0].astype(jnp.int32)
    dst = edge_index[1].astype(jnp.int32)
    pad = _E_PAD - _E
    src2 = jnp.concatenate([src, jnp.zeros((pad,), jnp.int32)])
    dst2 = jnp.concatenate([dst, jnp.full((pad,), _N, jnp.int32)])
    pvals = _edge_call(A, B, src2, dst2)
    pcnt = _cnt_call(dst2)
    return _combine(pvals, pcnt)
